# baseline (speedup 1.0000x reference)
import functools

import jax
import jax.numpy as jnp
from jax import lax
from jax.experimental import pallas as pl
from jax.experimental.pallas import tpu as pltpu
from jax.experimental.pallas import tpu_sc as plsc

NUM_CORES = 2
NUM_SUBCORES = 16
NUM_LANES = 16
NUM_WORKERS = NUM_CORES * NUM_SUBCORES
IDX_CHUNK = 128


def _mf_body(E, B_PER_W, u_hbm, v_hbm, ue_hbm, ie_hbm, ub_hbm, ib_hbm,
             out_hbm, idx_u, idx_v, urows, vrows, bias_u, bias_v, out_v, sem):
    wid = lax.axis_index("s") * NUM_CORES + lax.axis_index("c")
    base = wid * B_PER_W
    n_chunks = B_PER_W // IDX_CHUNK

    copies = []
    for j in range(n_chunks):
        src = pl.ds(base + j * IDX_CHUNK, IDX_CHUNK)
        copies.append(pltpu.make_async_copy(u_hbm.at[src], idx_u.at[j], sem))
        copies.append(pltpu.make_async_copy(v_hbm.at[src], idx_v.at[j], sem))
    for c in copies:
        c.start()
    for c in copies:
        c.wait()

    gathers = []
    for j in range(n_chunks):
        dst = pl.ds(j * IDX_CHUNK, IDX_CHUNK)
        gathers.append(pltpu.make_async_copy(ue_hbm.at[idx_u.at[j]], urows.at[dst], sem))
        gathers.append(pltpu.make_async_copy(ie_hbm.at[idx_v.at[j]], vrows.at[dst], sem))
        gathers.append(pltpu.make_async_copy(ub_hbm.at[idx_u.at[j]], bias_u.at[dst], sem))
        gathers.append(pltpu.make_async_copy(ib_hbm.at[idx_v.at[j]], bias_v.at[dst], sem))
    for g in gathers:
        g.start()
    for g in gathers:
        g.wait()

    lane = lax.iota(jnp.int32, NUM_LANES)

    def chunk_body(c, carry):
        rows = lane + c * NUM_LANES
        sl = pl.ds(c * NUM_LANES, NUM_LANES)
        acc = bias_u[sl] + bias_v[sl]
        for j in range(E):
            col = jnp.full((NUM_LANES,), j, jnp.int32)
            uj = plsc.load_gather(urows, [rows, col])
            vj = plsc.load_gather(vrows, [rows, col])
            acc = acc + uj * vj
        out_v[pl.ds(c * NUM_LANES, NUM_LANES)] = acc
        return carry

    lax.fori_loop(0, B_PER_W // NUM_LANES, chunk_body, 0)

    out_copy = pltpu.make_async_copy(out_v, out_hbm.at[pl.ds(base, B_PER_W)], sem)
    out_copy.start()
    out_copy.wait()


@jax.jit
def _mf_sc(u, v, user_emb, item_emb, user_bias, item_bias):
    B = u.shape[0]
    E = user_emb.shape[1]
    b_per_w = B // NUM_WORKERS
    n_chunks = b_per_w // IDX_CHUNK
    mesh = plsc.VectorSubcoreMesh(core_axis_name="c", subcore_axis_name="s")
    kern = pl.kernel(
        functools.partial(_mf_body, E, b_per_w),
        mesh=mesh,
        compiler_params=pltpu.CompilerParams(
            needs_layout_passes=False, use_tc_tiling_on_sc=False),
        out_type=jax.ShapeDtypeStruct((B,), jnp.float32),
        scratch_types=[
            pltpu.VMEM((n_chunks, IDX_CHUNK), jnp.int32),
            pltpu.VMEM((n_chunks, IDX_CHUNK), jnp.int32),
            pltpu.VMEM((b_per_w, E), jnp.float32),
            pltpu.VMEM((b_per_w, E), jnp.float32),
            pltpu.VMEM((b_per_w,), jnp.float32),
            pltpu.VMEM((b_per_w,), jnp.float32),
            pltpu.VMEM((b_per_w,), jnp.float32),
            pltpu.SemaphoreType.DMA,
        ],
    )
    return kern(u, v, user_emb, item_emb, user_bias, item_bias)


def kernel(u, v, user_emb, item_emb, user_bias, item_bias):
    u = u.astype(jnp.int32)
    v = v.astype(jnp.int32)
    ub = user_bias.reshape(-1)
    ib = item_bias.reshape(-1)
    return _mf_sc(u, v, user_emb, item_emb, ub, ib)
